# Initial kernel scaffold; baseline (speedup 1.0000x reference)
import jax
import jax.numpy as jnp
from jax.experimental import pallas as pl


def kernel(node_features, adjacency, W_t, W_u, b_u):
    raise NotImplementedError("write your pallas kernel here")


# speedup vs baseline: 1.8571x; 1.8571x over previous
import functools

import jax
import jax.numpy as jnp
from jax.experimental import pallas as pl
from jax.experimental.pallas import tpu as pltpu


def _transform_body(x_ref, wt_ref, t_ref):
    t_ref[:] = jnp.dot(x_ref[:], wt_ref[:], preferred_element_type=jnp.float32)


def _main_body(a_ref, t_ref, x_ref, wu_ref, bu_ref, out_ref, *, d):
    a = a_ref[:]
    acc = jnp.dot(a, t_ref[:], preferred_element_type=jnp.float32)
    deg = jnp.sum(a, axis=1, keepdims=True)
    dinv = jnp.where(deg == 0.0, 0.0, 1.0 / deg)
    nb = acc * dinv
    wu = wu_ref[:]
    out = (jnp.dot(x_ref[:], wu[:d, :], preferred_element_type=jnp.float32)
           + jnp.dot(nb, wu[d:, :], preferred_element_type=jnp.float32)
           + bu_ref[:])
    out_ref[:] = jnp.maximum(out, 0.0)


@jax.jit
def kernel(node_features, adjacency, W_t, W_u, b_u):
    n, d = node_features.shape
    units = W_t.shape[1]

    transformed = pl.pallas_call(
        _transform_body,
        out_shape=jax.ShapeDtypeStruct((n, units), jnp.float32),
    )(node_features, W_t)

    bm = 400
    nm = n // bm

    out = pl.pallas_call(
        functools.partial(_main_body, d=d),
        grid=(nm,),
        in_specs=[
            pl.BlockSpec((bm, n), lambda i: (i, 0)),
            pl.BlockSpec((n, units), lambda i: (0, 0)),
            pl.BlockSpec((bm, d), lambda i: (i, 0)),
            pl.BlockSpec((d + units, units), lambda i: (0, 0)),
            pl.BlockSpec((1, units), lambda i: (0, 0)),
        ],
        out_specs=pl.BlockSpec((bm, units), lambda i: (i, 0)),
        out_shape=jax.ShapeDtypeStruct((n, units), jnp.float32),
        compiler_params=pltpu.CompilerParams(
            dimension_semantics=("arbitrary",),
        ),
    )(adjacency, transformed, node_features, W_u, b_u.reshape(1, units))
    return out
